# Initial kernel scaffold; baseline (speedup 1.0000x reference)
import jax
import jax.numpy as jnp
from jax.experimental import pallas as pl


def kernel(Q, K, V, WQ_kernel, WQ_bias, WK_kernel, WK_bias, WV_kernel, WV_bias):
    raise NotImplementedError("write your pallas kernel here")


# speedup vs baseline: 72.0042x; 72.0042x over previous
import functools
import math

import jax
import jax.numpy as jnp
import numpy as np
from jax.experimental import pallas as pl
from jax.experimental.pallas import tpu as pltpu

R = 64


def _fft_consts(L):
    n = np.arange(R)
    F = np.exp(-2j * np.pi * np.outer(n, n) / R)
    T = np.exp(-2j * np.pi * np.outer(n, n) / L)
    return (F.real.astype(np.float32), F.imag.astype(np.float32),
            T.real.astype(np.float32), T.imag.astype(np.float32))


def _mm(a, b):
    return jnp.dot(a, b, preferred_element_type=jnp.float32,
                   precision=jax.lax.Precision.HIGHEST)


def _mm_fast(a, b):
    return jnp.dot(a, b, preferred_element_type=jnp.float32)


def _fft_fwd_real(x, fr, fi, tr, ti, L, w):
    xf = x.reshape(R, R * w)
    gr = _mm(fr, xf)
    gi = _mm(fi, xf)
    g3r = gr.reshape(R, R, w)
    g3i = gi.reshape(R, R, w)
    hr = g3r * tr[:, :, None] - g3i * ti[:, :, None]
    hi = g3r * ti[:, :, None] + g3i * tr[:, :, None]
    htr = jnp.swapaxes(hr, 0, 1).reshape(R, R * w)
    hti = jnp.swapaxes(hi, 0, 1).reshape(R, R * w)
    outr = _mm(fr, htr) - _mm(fi, hti)
    outi = _mm(fr, hti) + _mm(fi, htr)
    return outr.reshape(L, w), outi.reshape(L, w)


def _fft_cplx_realpart(xr, xi, fr, fi, tr, ti, L, w):
    xfr = xr.reshape(R, R * w)
    xfi = xi.reshape(R, R * w)
    gr = _mm(fr, xfr) - _mm(fi, xfi)
    gi = _mm(fr, xfi) + _mm(fi, xfr)
    g3r = gr.reshape(R, R, w)
    g3i = gi.reshape(R, R, w)
    hr = g3r * tr[:, :, None] - g3i * ti[:, :, None]
    hi = g3r * ti[:, :, None] + g3i * tr[:, :, None]
    htr = jnp.swapaxes(hr, 0, 1).reshape(R, R * w)
    hti = jnp.swapaxes(hi, 0, 1).reshape(R, R * w)
    outr = _mm(fr, htr) - _mm(fi, hti)
    return outr.reshape(L, w)


def _corr_body(fr_ref, fi_ref, tr_ref, ti_ref, x_ref, y_ref, o_ref, *, L, w):
    fr = fr_ref[...]
    fi = fi_ref[...]
    tr = tr_ref[...]
    ti = ti_ref[...]
    qr, qi = _fft_fwd_real(x_ref[...], fr, fi, tr, ti, L, w)
    kr, ki = _fft_fwd_real(y_ref[...], fr, fi, tr, ti, L, w)
    pr = qr * kr + qi * ki
    pi = qr * ki - qi * kr
    o_ref[...] = _fft_cplx_realpart(pr, pi, fr, fi, tr, ti, L, w) * (1.0 / L)


def _corr_call(x, y, dblk):
    B, L, D = x.shape
    frn, fin, trn, tin = _fft_consts(L)
    grid = (B, D // dblk)
    const_spec = pl.BlockSpec((R, R), lambda b, j: (0, 0))
    data_spec = pl.BlockSpec((None, L, dblk), lambda b, j: (b, 0, j))
    return pl.pallas_call(
        functools.partial(_corr_body, L=L, w=dblk),
        grid=grid,
        in_specs=[const_spec, const_spec, const_spec, const_spec,
                  data_spec, data_spec],
        out_specs=data_spec,
        out_shape=jax.ShapeDtypeStruct((B, L, D), jnp.float32),
    )(jnp.asarray(frn), jnp.asarray(fin), jnp.asarray(trn), jnp.asarray(tin),
      x, y)


def _proj_body(x_ref, w_ref, b_ref, o_ref):
    o_ref[...] = _mm_fast(x_ref[...], w_ref[...]) + b_ref[...]


def _proj_call(x, w, b, lblk):
    B, L, DM = x.shape
    D = w.shape[1]
    grid = (B, L // lblk)
    return pl.pallas_call(
        _proj_body,
        grid=grid,
        in_specs=[
            pl.BlockSpec((None, lblk, DM), lambda bb, i: (bb, i, 0)),
            pl.BlockSpec((DM, D), lambda bb, i: (0, 0)),
            pl.BlockSpec((1, D), lambda bb, i: (0, 0)),
        ],
        out_specs=pl.BlockSpec((None, lblk, D), lambda bb, i: (bb, i, 0)),
        out_shape=jax.ShapeDtypeStruct((B, L, D), jnp.float32),
    )(x, w, b.reshape(1, D))


def _topk_body(rxx_ref, s_ref, *, L, k, w):
    x = rxx_ref[...]
    iota = jax.lax.broadcasted_iota(jnp.int32, (L, w), 0)
    neg = jnp.float32(-jnp.inf)
    vals = []
    idxs = []
    cur = x
    for _ in range(k):
        m = jnp.max(cur, axis=0)
        im = jnp.min(jnp.where(cur == m[None, :], iota, L), axis=0)
        vals.append(m)
        idxs.append(im)
        cur = jnp.where(iota == im[None, :], neg, cur)
    vs = jnp.stack(vals, axis=0)
    e = jnp.exp(vs - vs[0][None, :])
    wsm = e / jnp.sum(e, axis=0)[None, :]
    s = jnp.zeros((L, w), jnp.float32)
    for j in range(k):
        s = s + jnp.where(iota == idxs[j][None, :], wsm[j][None, :], 0.0)
    s_ref[...] = s


def _topk_call(rxx, k, dblk):
    B, L, D = rxx.shape
    grid = (B, D // dblk)
    spec = pl.BlockSpec((None, L, dblk), lambda b, j: (b, 0, j))
    return pl.pallas_call(
        functools.partial(_topk_body, L=L, k=k, w=dblk),
        grid=grid,
        in_specs=[spec],
        out_specs=spec,
        out_shape=jax.ShapeDtypeStruct((B, L, D), jnp.float32),
    )(rxx)


def kernel(Q, K, V, WQ_kernel, WQ_bias, WK_kernel, WK_bias, WV_kernel,
           WV_bias):
    B, L, DM = Q.shape
    S = K.shape[1]
    if S >= L:
        K = K[:, :L, :]
        V = V[:, :L, :]
    else:
        K = jnp.pad(K, ((0, 0), (0, L - S), (0, 0)))
        V = jnp.pad(V, ((0, 0), (0, L - S), (0, 0)))
    k = int(math.floor(2 * math.log(L)))

    Qp = _proj_call(Q, WQ_kernel, WQ_bias, lblk=512)
    Kp = _proj_call(K, WK_kernel, WK_bias, lblk=512)
    Vp = _proj_call(V, WV_kernel, WV_bias, lblk=512)

    Rxx = _corr_call(Qp, Kp, dblk=256)
    s = _topk_call(Rxx, k, dblk=256)
    A = _corr_call(Vp, s, dblk=256)
    return A


# speedup vs baseline: 76.1653x; 1.0578x over previous
import functools
import math

import jax
import jax.numpy as jnp
import numpy as np
from jax.experimental import pallas as pl

R = 64


def _fft_consts(L):
    n = np.arange(R)
    F = np.exp(-2j * np.pi * np.outer(n, n) / R)
    T = np.exp(-2j * np.pi * np.outer(n, n) / L)
    return (F.real.astype(np.float32), F.imag.astype(np.float32),
            T.real.astype(np.float32), T.imag.astype(np.float32))


def _mm_hi(a, b):
    return jnp.dot(a, b, preferred_element_type=jnp.float32,
                   precision=jax.lax.Precision.HIGHEST)


def _mm_mid(a, b):
    return _mm_hi(a, b)


def _mm_fast(a, b):
    return jnp.dot(a, b, preferred_element_type=jnp.float32)


def _fft_fwd_real(x, fr, fi, tr, ti, L, w, mm):
    xf = x.reshape(R, R * w)
    gr = mm(fr, xf)
    gi = mm(fi, xf)
    g3r = gr.reshape(R, R, w)
    g3i = gi.reshape(R, R, w)
    hr = g3r * tr[:, :, None] - g3i * ti[:, :, None]
    hi = g3r * ti[:, :, None] + g3i * tr[:, :, None]
    htr = jnp.swapaxes(hr, 0, 1).reshape(R, R * w)
    hti = jnp.swapaxes(hi, 0, 1).reshape(R, R * w)
    outr = mm(fr, htr) - mm(fi, hti)
    outi = mm(fr, hti) + mm(fi, htr)
    return outr.reshape(L, w), outi.reshape(L, w)


def _fft_cplx_realpart(xr, xi, fr, fi, tr, ti, L, w, mm):
    xfr = xr.reshape(R, R * w)
    xfi = xi.reshape(R, R * w)
    gr = mm(fr, xfr) - mm(fi, xfi)
    gi = mm(fr, xfi) + mm(fi, xfr)
    g3r = gr.reshape(R, R, w)
    g3i = gi.reshape(R, R, w)
    hr = g3r * tr[:, :, None] - g3i * ti[:, :, None]
    hi = g3r * ti[:, :, None] + g3i * tr[:, :, None]
    htr = jnp.swapaxes(hr, 0, 1).reshape(R, R * w)
    hti = jnp.swapaxes(hi, 0, 1).reshape(R, R * w)
    return (mm(fr, htr) - mm(fi, hti)).reshape(L, w)


def _topk_filter(x, L, k, w):
    iota = jax.lax.broadcasted_iota(jnp.int32, (L, w), 0)
    neg = jnp.float32(-jnp.inf)
    vals = []
    idxs = []
    cur = x
    for _ in range(k):
        m = jnp.max(cur, axis=0)
        im = jnp.min(jnp.where(cur == m[None, :], iota, L), axis=0)
        vals.append(m)
        idxs.append(im)
        cur = jnp.where(iota == im[None, :], neg, cur)
    vs = jnp.stack(vals, axis=0)
    e = jnp.exp(vs - vs[0][None, :])
    wsm = e / jnp.sum(e, axis=0)[None, :]
    s = jnp.zeros((L, w), jnp.float32)
    for j in range(k):
        s = s + jnp.where(iota == idxs[j][None, :], wsm[j][None, :], 0.0)
    return s


def _fused_body(fr_ref, fi_ref, tr_ref, ti_ref, qp_ref, kp_ref, vp_ref,
                o_ref, *, L, k, w):
    fr = fr_ref[...]
    fi = fi_ref[...]
    tr = tr_ref[...]
    ti = ti_ref[...]
    qr, qi = _fft_fwd_real(qp_ref[...], fr, fi, tr, ti, L, w, _mm_hi)
    kr, ki = _fft_fwd_real(kp_ref[...], fr, fi, tr, ti, L, w, _mm_hi)
    pr = qr * kr + qi * ki
    pi = qr * ki - qi * kr
    rxx = _fft_cplx_realpart(pr, pi, fr, fi, tr, ti, L, w, _mm_hi) * (1.0 / L)
    s = _topk_filter(rxx, L, k, w)
    vr, vi = _fft_fwd_real(vp_ref[...], fr, fi, tr, ti, L, w, _mm_mid)
    sr, si = _fft_fwd_real(s, fr, fi, tr, ti, L, w, _mm_mid)
    ar = vr * sr + vi * si
    ai = vr * si - vi * sr
    o_ref[...] = _fft_cplx_realpart(ar, ai, fr, fi, tr, ti, L, w,
                                    _mm_mid) * (1.0 / L)


def _fused_call(qp, kp, vp, k, dblk):
    B, L, D = qp.shape
    frn, fin, trn, tin = _fft_consts(L)
    grid = (B, D // dblk)
    const_spec = pl.BlockSpec((R, R), lambda b, j: (0, 0))
    data_spec = pl.BlockSpec((None, L, dblk), lambda b, j: (b, 0, j))
    return pl.pallas_call(
        functools.partial(_fused_body, L=L, k=k, w=dblk),
        grid=grid,
        in_specs=[const_spec, const_spec, const_spec, const_spec,
                  data_spec, data_spec, data_spec],
        out_specs=data_spec,
        out_shape=jax.ShapeDtypeStruct((B, L, D), jnp.float32),
    )(jnp.asarray(frn), jnp.asarray(fin), jnp.asarray(trn), jnp.asarray(tin),
      qp, kp, vp)


def _proj_body(x_ref, w_ref, b_ref, o_ref):
    o_ref[...] = _mm_fast(x_ref[...], w_ref[...]) + b_ref[...]


def _proj_call(x, w, b, lblk):
    B, L, DM = x.shape
    D = w.shape[1]
    grid = (B, L // lblk)
    return pl.pallas_call(
        _proj_body,
        grid=grid,
        in_specs=[
            pl.BlockSpec((None, lblk, DM), lambda bb, i: (bb, i, 0)),
            pl.BlockSpec((DM, D), lambda bb, i: (0, 0)),
            pl.BlockSpec((1, D), lambda bb, i: (0, 0)),
        ],
        out_specs=pl.BlockSpec((None, lblk, D), lambda bb, i: (bb, i, 0)),
        out_shape=jax.ShapeDtypeStruct((B, L, D), jnp.float32),
    )(x, w, b.reshape(1, D))


def kernel(Q, K, V, WQ_kernel, WQ_bias, WK_kernel, WK_bias, WV_kernel,
           WV_bias):
    B, L, DM = Q.shape
    S = K.shape[1]
    if S >= L:
        K = K[:, :L, :]
        V = V[:, :L, :]
    else:
        K = jnp.pad(K, ((0, 0), (0, L - S), (0, 0)))
        V = jnp.pad(V, ((0, 0), (0, L - S), (0, 0)))
    k = int(math.floor(2 * math.log(L)))

    Qp = _proj_call(Q, WQ_kernel, WQ_bias, lblk=512)
    Kp = _proj_call(K, WK_kernel, WK_bias, lblk=512)
    Vp = _proj_call(V, WV_kernel, WV_bias, lblk=512)
    return _fused_call(Qp, Kp, Vp, k, dblk=128)


# speedup vs baseline: 86.0306x; 1.1295x over previous
import functools
import math

import jax
import jax.numpy as jnp
import numpy as np
from jax.experimental import pallas as pl

R = 64


def _fft_consts(L):
    n = np.arange(R)
    F = np.exp(-2j * np.pi * np.outer(n, n) / R)
    T = np.exp(-2j * np.pi * np.outer(n, n) / L)
    return (F.real.astype(np.float32), F.imag.astype(np.float32),
            T.real.astype(np.float32), T.imag.astype(np.float32))


def _mm_hi(a, b):
    return jnp.dot(a, b, preferred_element_type=jnp.float32,
                   precision=jax.lax.Precision.HIGHEST)


def _mm_fast(a, b):
    return jnp.dot(a, b, preferred_element_type=jnp.float32)


def _split(x):
    hi = x.astype(jnp.bfloat16)
    lo = (x - hi.astype(jnp.float32)).astype(jnp.bfloat16)
    return hi, lo


def _mm3(a2, b2):
    ah, al = a2
    bh, bl = b2
    return (_mm_fast(ah, bh) + _mm_fast(ah, bl)) + _mm_fast(al, bh)


def _fft_fwd_real(x, fr2, fi2, tr, ti, L, w):
    xf2 = _split(x.reshape(R, R * w))
    gr = _mm3(fr2, xf2)
    gi = _mm3(fi2, xf2)
    g3r = gr.reshape(R, R, w)
    g3i = gi.reshape(R, R, w)
    hr = g3r * tr[:, :, None] - g3i * ti[:, :, None]
    hi = g3r * ti[:, :, None] + g3i * tr[:, :, None]
    htr2 = _split(jnp.swapaxes(hr, 0, 1).reshape(R, R * w))
    hti2 = _split(jnp.swapaxes(hi, 0, 1).reshape(R, R * w))
    outr = _mm3(fr2, htr2) - _mm3(fi2, hti2)
    outi = _mm3(fr2, hti2) + _mm3(fi2, htr2)
    return outr.reshape(L, w), outi.reshape(L, w)


def _fft_cplx_realpart(xr, xi, fr2, fi2, tr, ti, L, w):
    xfr2 = _split(xr.reshape(R, R * w))
    xfi2 = _split(xi.reshape(R, R * w))
    gr = _mm3(fr2, xfr2) - _mm3(fi2, xfi2)
    gi = _mm3(fr2, xfi2) + _mm3(fi2, xfr2)
    g3r = gr.reshape(R, R, w)
    g3i = gi.reshape(R, R, w)
    hr = g3r * tr[:, :, None] - g3i * ti[:, :, None]
    hi = g3r * ti[:, :, None] + g3i * tr[:, :, None]
    htr2 = _split(jnp.swapaxes(hr, 0, 1).reshape(R, R * w))
    hti2 = _split(jnp.swapaxes(hi, 0, 1).reshape(R, R * w))
    return (_mm3(fr2, htr2) - _mm3(fi2, hti2)).reshape(L, w)


def _topk_filter(x, L, k, w):
    iota = jax.lax.broadcasted_iota(jnp.int32, (L, w), 0)
    neg = jnp.float32(-jnp.inf)
    vals = []
    idxs = []
    cur = x
    for _ in range(k):
        m = jnp.max(cur, axis=0)
        im = jnp.min(jnp.where(cur == m[None, :], iota, L), axis=0)
        vals.append(m)
        idxs.append(im)
        cur = jnp.where(iota == im[None, :], neg, cur)
    vs = jnp.stack(vals, axis=0)
    e = jnp.exp(vs - vs[0][None, :])
    wsm = e / jnp.sum(e, axis=0)[None, :]
    s = jnp.zeros((L, w), jnp.float32)
    for j in range(k):
        s = s + jnp.where(iota == idxs[j][None, :], wsm[j][None, :], 0.0)
    return s


def _fused_body(fr_ref, fi_ref, tr_ref, ti_ref, qp_ref, kp_ref, vp_ref,
                o_ref, *, L, k, w):
    fr2 = _split(fr_ref[...])
    fi2 = _split(fi_ref[...])
    tr = tr_ref[...]
    ti = ti_ref[...]
    qr, qi = _fft_fwd_real(qp_ref[...], fr2, fi2, tr, ti, L, w)
    kr, ki = _fft_fwd_real(kp_ref[...], fr2, fi2, tr, ti, L, w)
    pr = qr * kr + qi * ki
    pi = qr * ki - qi * kr
    rxx = _fft_cplx_realpart(pr, pi, fr2, fi2, tr, ti, L, w) * (1.0 / L)
    s = _topk_filter(rxx, L, k, w)
    vr, vi = _fft_fwd_real(vp_ref[...], fr2, fi2, tr, ti, L, w)
    sr, si = _fft_fwd_real(s, fr2, fi2, tr, ti, L, w)
    ar = vr * sr + vi * si
    ai = vr * si - vi * sr
    o_ref[...] = _fft_cplx_realpart(ar, ai, fr2, fi2, tr, ti, L, w) * (1.0 / L)


def _fused_call(qp, kp, vp, k, dblk):
    B, L, D = qp.shape
    frn, fin, trn, tin = _fft_consts(L)
    grid = (B, D // dblk)
    const_spec = pl.BlockSpec((R, R), lambda b, j: (0, 0))
    data_spec = pl.BlockSpec((None, L, dblk), lambda b, j: (b, 0, j))
    return pl.pallas_call(
        functools.partial(_fused_body, L=L, k=k, w=dblk),
        grid=grid,
        in_specs=[const_spec, const_spec, const_spec, const_spec,
                  data_spec, data_spec, data_spec],
        out_specs=data_spec,
        out_shape=jax.ShapeDtypeStruct((B, L, D), jnp.float32),
    )(jnp.asarray(frn), jnp.asarray(fin), jnp.asarray(trn), jnp.asarray(tin),
      qp, kp, vp)


def _proj_body(x_ref, w_ref, b_ref, o_ref):
    o_ref[...] = _mm_fast(x_ref[...], w_ref[...]) + b_ref[...]


def _proj_call(x, w, b, lblk):
    B, L, DM = x.shape
    D = w.shape[1]
    grid = (B, L // lblk)
    return pl.pallas_call(
        _proj_body,
        grid=grid,
        in_specs=[
            pl.BlockSpec((None, lblk, DM), lambda bb, i: (bb, i, 0)),
            pl.BlockSpec((DM, D), lambda bb, i: (0, 0)),
            pl.BlockSpec((1, D), lambda bb, i: (0, 0)),
        ],
        out_specs=pl.BlockSpec((None, lblk, D), lambda bb, i: (bb, i, 0)),
        out_shape=jax.ShapeDtypeStruct((B, L, D), jnp.float32),
    )(x, w, b.reshape(1, D))


def kernel(Q, K, V, WQ_kernel, WQ_bias, WK_kernel, WK_bias, WV_kernel,
           WV_bias):
    B, L, DM = Q.shape
    S = K.shape[1]
    if S >= L:
        K = K[:, :L, :]
        V = V[:, :L, :]
    else:
        K = jnp.pad(K, ((0, 0), (0, L - S), (0, 0)))
        V = jnp.pad(V, ((0, 0), (0, L - S), (0, 0)))
    k = int(math.floor(2 * math.log(L)))

    Qp = _proj_call(Q, WQ_kernel, WQ_bias, lblk=512)
    Kp = _proj_call(K, WK_kernel, WK_bias, lblk=512)
    Vp = _proj_call(V, WV_kernel, WV_bias, lblk=512)
    return _fused_call(Qp, Kp, Vp, k, dblk=128)
